# Initial kernel scaffold; baseline (speedup 1.0000x reference)
import jax
import jax.numpy as jnp
from jax.experimental import pallas as pl


def kernel(query, keys, train):
    raise NotImplementedError("write your pallas kernel here")


# speedup vs baseline: 8.0763x; 8.0763x over previous
import jax
import jax.numpy as jnp
from jax.experimental import pallas as pl

_N = 1024
_M = 512
_D = 256
_BN = 128


def _hotmap_kernel(q_ref, k_ref, out_ref):
    q = q_ref[...]
    k = k_ref[...]

    norm = jnp.sqrt(jnp.sum(q * q, axis=1, keepdims=True))
    qn = q / jnp.maximum(norm, 1e-12)

    ones = jnp.ones((1, _D), jnp.float32)
    ksq = jax.lax.dot_general(
        ones, k * k, (((1,), (1,)), ((), ())),
        preferred_element_type=jnp.float32, precision=jax.lax.Precision.HIGHEST,
    )

    qk = jax.lax.dot_general(
        qn, k, (((1,), (1,)), ((), ())), preferred_element_type=jnp.float32,
        precision=jax.lax.Precision.HIGHEST,
    )
    dist = ksq - 2.0 * qk

    col = jax.lax.broadcasted_iota(jnp.int32, (_BN, _M), 1)
    best_d = None
    for _ in range(4):
        dmin = jnp.min(dist, axis=1, keepdims=True)
        idx_j = jnp.min(jnp.where(dist <= dmin, col, _M), axis=1, keepdims=True)
        hit = col == idx_j
        dist = jnp.where(hit, jnp.inf, dist)
        g_j = jax.lax.dot_general(
            hit.astype(jnp.float32), k, (((1,), (0,)), ((), ())),
            preferred_element_type=jnp.float32,
            precision=jax.lax.Precision.HIGHEST,
        )
        diff_j = qn - g_j
        d_j = jnp.sum(diff_j * diff_j, axis=1, keepdims=True)
        if best_d is None:
            best_d, best_idx, best_g = d_j, idx_j, g_j
        else:
            take = (d_j < best_d) | ((d_j == best_d) & (idx_j < best_idx))
            best_d = jnp.where(take, d_j, best_d)
            best_idx = jnp.where(take, idx_j, best_idx)
            best_g = jnp.where(take, g_j, best_g)

    diff = qn - best_g
    d2 = diff * diff
    out_ref[...] = jnp.sum(d2 * d2, axis=1, keepdims=True)


def kernel(query, keys, train):
    q = query[0]
    b, c, h, w = q.shape
    qr = jnp.transpose(q, (0, 2, 3, 1)).reshape(b * h * w, c)
    loss = pl.pallas_call(
        _hotmap_kernel,
        grid=(_N // _BN,),
        in_specs=[
            pl.BlockSpec((_BN, _D), lambda i: (i, 0)),
            pl.BlockSpec((_M, _D), lambda i: (0, 0)),
        ],
        out_specs=pl.BlockSpec((_BN, 1), lambda i: (i, 0)),
        out_shape=jax.ShapeDtypeStruct((_N, 1), jnp.float32),
    )(qr, keys[0])
    return loss.reshape(b, h, w, 1)


# speedup vs baseline: 9.2907x; 1.1504x over previous
import jax
import jax.numpy as jnp
from jax.experimental import pallas as pl

_N = 1024
_M = 512
_D = 256
_BN = 128


def _sel(onehot, part):
    return jax.lax.dot_general(
        onehot, part, (((1,), (0,)), ((), ())),
        preferred_element_type=jnp.float32,
    )


def _hotmap_kernel(q_ref, k_ref, khi_ref, kmid_ref, klo_ref, out_ref):
    q = q_ref[...]
    k = k_ref[...]

    norm = jnp.sqrt(jnp.sum(q * q, axis=1, keepdims=True))
    qn = q / jnp.maximum(norm, 1e-12)

    ones = jnp.ones((1, _D), jnp.float32)
    ksq = jax.lax.dot_general(
        ones, k * k, (((1,), (1,)), ((), ())),
        preferred_element_type=jnp.float32, precision=jax.lax.Precision.HIGHEST,
    )

    qk = jax.lax.dot_general(
        qn, k, (((1,), (1,)), ((), ())), preferred_element_type=jnp.float32,
        precision=jax.lax.Precision.HIGHEST,
    )
    dist = ksq - 2.0 * qk

    col = jax.lax.broadcasted_iota(jnp.int32, (_BN, _M), 1)
    dmin1 = jnp.min(dist, axis=1, keepdims=True)
    idx1 = jnp.min(jnp.where(dist <= dmin1, col, _M), axis=1, keepdims=True)
    hit1 = col == idx1
    dist2 = jnp.where(hit1, jnp.inf, dist)
    dmin2 = jnp.min(dist2, axis=1, keepdims=True)
    idx2 = jnp.min(jnp.where(dist2 <= dmin2, col, _M), axis=1, keepdims=True)
    hit2 = col == idx2

    khi = khi_ref[...]
    kmid = kmid_ref[...]
    klo = klo_ref[...]
    oh1 = hit1.astype(jnp.float32)
    oh2 = hit2.astype(jnp.float32)
    g1 = _sel(oh1, khi) + _sel(oh1, kmid) + _sel(oh1, klo)
    g2 = _sel(oh2, khi) + _sel(oh2, kmid) + _sel(oh2, klo)

    e1 = qn - g1
    e2 = qn - g2
    d1 = jnp.sum(e1 * e1, axis=1, keepdims=True)
    d2 = jnp.sum(e2 * e2, axis=1, keepdims=True)
    take2 = (d2 < d1) | ((d2 == d1) & (idx2 < idx1))
    diff = jnp.where(take2, e2, e1)
    d2q = diff * diff
    out_ref[...] = jnp.sum(d2q * d2q, axis=1, keepdims=True)


def kernel(query, keys, train):
    q = query[0]
    b, c, h, w = q.shape
    qr = jnp.transpose(q, (0, 2, 3, 1)).reshape(b * h * w, c)
    k = keys[0]
    mask = jnp.uint32(0xFFFF0000)
    khi = jax.lax.bitcast_convert_type(
        jax.lax.bitcast_convert_type(k, jnp.uint32) & mask, jnp.float32)
    r1 = k - khi
    kmid = jax.lax.bitcast_convert_type(
        jax.lax.bitcast_convert_type(r1, jnp.uint32) & mask, jnp.float32)
    klo = r1 - kmid
    full = pl.BlockSpec((_M, _D), lambda i: (0, 0))
    loss = pl.pallas_call(
        _hotmap_kernel,
        grid=(_N // _BN,),
        in_specs=[pl.BlockSpec((_BN, _D), lambda i: (i, 0)), full, full, full, full],
        out_specs=pl.BlockSpec((_BN, 1), lambda i: (i, 0)),
        out_shape=jax.ShapeDtypeStruct((_N, 1), jnp.float32),
    )(qr, k, khi, kmid, klo)
    return loss.reshape(b, h, w, 1)


# speedup vs baseline: 18.0358x; 1.9413x over previous
import jax
import jax.numpy as jnp
from jax.experimental import pallas as pl
from jax.experimental.pallas import tpu as pltpu

_N = 1024
_M = 512
_D = 256
_BN = 256


def _sel(onehot, part):
    return jax.lax.dot_general(
        onehot, part, (((1,), (0,)), ((), ())),
        preferred_element_type=jnp.float32,
    )


def _bf16_exact_part(x):
    u = jax.lax.bitcast_convert_type(x, jnp.uint32)
    return jax.lax.bitcast_convert_type(u & jnp.uint32(0xFFFF0000), jnp.float32)


def _hotmap_kernel(q_ref, k_ref, out_ref, ksq_ref, khi_ref, kmid_ref, klo_ref):
    k = k_ref[...]

    @pl.when(pl.program_id(0) == 0)
    def _init():
        ones = jnp.ones((1, _D), jnp.float32)
        ksq_ref[...] = jax.lax.dot_general(
            ones, k * k, (((1,), (1,)), ((), ())),
            preferred_element_type=jnp.float32,
            precision=jax.lax.Precision.HIGHEST,
        )
        khi = _bf16_exact_part(k)
        r1 = k - khi
        kmid = _bf16_exact_part(r1)
        khi_ref[...] = khi
        kmid_ref[...] = kmid
        klo_ref[...] = r1 - kmid

    q = q_ref[...]

    norm = jnp.sqrt(jnp.sum(q * q, axis=1, keepdims=True))
    qn = q / jnp.maximum(norm, 1e-12)

    qk = jax.lax.dot_general(
        qn, k, (((1,), (1,)), ((), ())), preferred_element_type=jnp.float32,
        precision=jax.lax.Precision.HIGHEST,
    )
    dist = ksq_ref[...] - 2.0 * qk

    col = jax.lax.broadcasted_iota(jnp.int32, (_BN, _M), 1)
    dmin1 = jnp.min(dist, axis=1, keepdims=True)
    idx1 = jnp.min(jnp.where(dist <= dmin1, col, _M), axis=1, keepdims=True)
    hit1 = col == idx1
    dist2 = jnp.where(hit1, jnp.inf, dist)
    dmin2 = jnp.min(dist2, axis=1, keepdims=True)
    idx2 = jnp.min(jnp.where(dist2 <= dmin2, col, _M), axis=1, keepdims=True)
    hit2 = col == idx2

    khi = khi_ref[...]
    kmid = kmid_ref[...]
    klo = klo_ref[...]
    oh1 = hit1.astype(jnp.float32)
    oh2 = hit2.astype(jnp.float32)
    g1 = _sel(oh1, khi) + _sel(oh1, kmid) + _sel(oh1, klo)
    g2 = _sel(oh2, khi) + _sel(oh2, kmid) + _sel(oh2, klo)

    e1 = qn - g1
    e2 = qn - g2
    d1 = jnp.sum(e1 * e1, axis=1, keepdims=True)
    d2 = jnp.sum(e2 * e2, axis=1, keepdims=True)
    take2 = (d2 < d1) | ((d2 == d1) & (idx2 < idx1))
    diff = jnp.where(take2, e2, e1)
    d2q = diff * diff
    out_ref[...] = jnp.sum(d2q * d2q, axis=1, keepdims=True)


def kernel(query, keys, train):
    q = query[0]
    b, c, h, w = q.shape
    qr = jnp.transpose(q, (0, 2, 3, 1)).reshape(b * h * w, c)
    loss = pl.pallas_call(
        _hotmap_kernel,
        grid=(_N // _BN,),
        in_specs=[
            pl.BlockSpec((_BN, _D), lambda i: (i, 0)),
            pl.BlockSpec((_M, _D), lambda i: (0, 0)),
        ],
        out_specs=pl.BlockSpec((_BN, 1), lambda i: (i, 0)),
        out_shape=jax.ShapeDtypeStruct((_N, 1), jnp.float32),
        scratch_shapes=[
            pltpu.VMEM((1, _M), jnp.float32),
            pltpu.VMEM((_M, _D), jnp.float32),
            pltpu.VMEM((_M, _D), jnp.float32),
            pltpu.VMEM((_M, _D), jnp.float32),
        ],
    )(qr, keys[0])
    return loss.reshape(b, h, w, 1)


# speedup vs baseline: 20.9885x; 1.1637x over previous
import jax
import jax.numpy as jnp
from jax.experimental import pallas as pl
from jax.experimental.pallas import tpu as pltpu

_N = 1024
_M = 512
_D = 256
_BN = 512


def _sel(onehot, part):
    return jax.lax.dot_general(
        onehot, part, (((1,), (0,)), ((), ())),
        preferred_element_type=jnp.float32,
    )


def _bf16_exact_part(x):
    u = jax.lax.bitcast_convert_type(x, jnp.uint32)
    return jax.lax.bitcast_convert_type(u & jnp.uint32(0xFFFF0000), jnp.float32)


def _hotmap_kernel(q_ref, k_ref, out_ref, ksq_ref, khi_ref, kmid_ref, klo_ref):
    k = k_ref[...]

    @pl.when(pl.program_id(0) == 0)
    def _init():
        ones = jnp.ones((1, _D), jnp.float32)
        ksq_ref[...] = jax.lax.dot_general(
            ones, k * k, (((1,), (1,)), ((), ())),
            preferred_element_type=jnp.float32,
            precision=jax.lax.Precision.HIGHEST,
        )
        khi = _bf16_exact_part(k)
        r1 = k - khi
        kmid = _bf16_exact_part(r1)
        khi_ref[...] = khi
        kmid_ref[...] = kmid
        klo_ref[...] = r1 - kmid

    q = q_ref[...]

    norm = jnp.sqrt(jnp.sum(q * q, axis=1, keepdims=True))
    qn = q / jnp.maximum(norm, 1e-12)

    qk = jax.lax.dot_general(
        qn, k, (((1,), (1,)), ((), ())), preferred_element_type=jnp.float32,
        precision=jax.lax.Precision.HIGHEST,
    )
    dist = ksq_ref[...] - 2.0 * qk

    col = jax.lax.broadcasted_iota(jnp.int32, (_BN, _M), 1)
    dmin1 = jnp.min(dist, axis=1, keepdims=True)
    idx1 = jnp.min(jnp.where(dist <= dmin1, col, _M), axis=1, keepdims=True)
    hit1 = col == idx1
    dist2 = jnp.where(hit1, jnp.inf, dist)
    dmin2 = jnp.min(dist2, axis=1, keepdims=True)
    idx2 = jnp.min(jnp.where(dist2 <= dmin2, col, _M), axis=1, keepdims=True)
    hit2 = col == idx2

    khi = khi_ref[...]
    kmid = kmid_ref[...]
    klo = klo_ref[...]
    oh1 = hit1.astype(jnp.float32)
    oh2 = hit2.astype(jnp.float32)
    g1 = _sel(oh1, khi) + _sel(oh1, kmid) + _sel(oh1, klo)
    g2 = _sel(oh2, khi) + _sel(oh2, kmid) + _sel(oh2, klo)

    e1 = qn - g1
    e2 = qn - g2
    d1 = jnp.sum(e1 * e1, axis=1, keepdims=True)
    d2 = jnp.sum(e2 * e2, axis=1, keepdims=True)
    take2 = (d2 < d1) | ((d2 == d1) & (idx2 < idx1))
    diff = jnp.where(take2, e2, e1)
    d2q = diff * diff
    out_ref[...] = jnp.sum(d2q * d2q, axis=1, keepdims=True)


def kernel(query, keys, train):
    q = query[0]
    b, c, h, w = q.shape
    qr = jnp.transpose(q, (0, 2, 3, 1)).reshape(b * h * w, c)
    loss = pl.pallas_call(
        _hotmap_kernel,
        grid=(_N // _BN,),
        in_specs=[
            pl.BlockSpec((_BN, _D), lambda i: (i, 0)),
            pl.BlockSpec((_M, _D), lambda i: (0, 0)),
        ],
        out_specs=pl.BlockSpec((_BN, 1), lambda i: (i, 0)),
        out_shape=jax.ShapeDtypeStruct((_N, 1), jnp.float32),
        scratch_shapes=[
            pltpu.VMEM((1, _M), jnp.float32),
            pltpu.VMEM((_M, _D), jnp.float32),
            pltpu.VMEM((_M, _D), jnp.float32),
            pltpu.VMEM((_M, _D), jnp.float32),
        ],
    )(qr, keys[0])
    return loss.reshape(b, h, w, 1)


# speedup vs baseline: 22.6483x; 1.0791x over previous
import jax
import jax.numpy as jnp
from jax.experimental import pallas as pl
from jax.experimental.pallas import tpu as pltpu

_N = 1024
_M = 512
_D = 256
_BN = 1024


def _sel(onehot, part):
    return jax.lax.dot_general(
        onehot, part, (((1,), (0,)), ((), ())),
        preferred_element_type=jnp.float32,
    )


def _bf16_exact_part(x):
    u = jax.lax.bitcast_convert_type(x, jnp.uint32)
    return jax.lax.bitcast_convert_type(u & jnp.uint32(0xFFFF0000), jnp.float32)


def _hotmap_kernel(q_ref, k_ref, out_ref, ksq_ref, khi_ref, kmid_ref, klo_ref):
    k = k_ref[...]

    @pl.when(pl.program_id(0) == 0)
    def _init():
        ones = jnp.ones((1, _D), jnp.float32)
        ksq_ref[...] = jax.lax.dot_general(
            ones, k * k, (((1,), (1,)), ((), ())),
            preferred_element_type=jnp.float32,
            precision=jax.lax.Precision.HIGHEST,
        )
        khi = _bf16_exact_part(k)
        r1 = k - khi
        kmid = _bf16_exact_part(r1)
        khi_ref[...] = khi
        kmid_ref[...] = kmid
        klo_ref[...] = r1 - kmid

    q = q_ref[...]

    norm = jnp.sqrt(jnp.sum(q * q, axis=1, keepdims=True))
    qn = q / jnp.maximum(norm, 1e-12)

    qk = jax.lax.dot_general(
        qn, k, (((1,), (1,)), ((), ())), preferred_element_type=jnp.float32,
        precision=jax.lax.Precision.HIGHEST,
    )
    dist = ksq_ref[...] - 2.0 * qk

    col = jax.lax.broadcasted_iota(jnp.int32, (_BN, _M), 1)
    dmin1 = jnp.min(dist, axis=1, keepdims=True)
    idx1 = jnp.min(jnp.where(dist <= dmin1, col, _M), axis=1, keepdims=True)
    hit1 = col == idx1
    dist2 = jnp.where(hit1, jnp.inf, dist)
    dmin2 = jnp.min(dist2, axis=1, keepdims=True)
    idx2 = jnp.min(jnp.where(dist2 <= dmin2, col, _M), axis=1, keepdims=True)
    hit2 = col == idx2

    khi = khi_ref[...]
    kmid = kmid_ref[...]
    klo = klo_ref[...]
    oh1 = hit1.astype(jnp.float32)
    oh2 = hit2.astype(jnp.float32)
    g1 = _sel(oh1, khi) + _sel(oh1, kmid) + _sel(oh1, klo)
    g2 = _sel(oh2, khi) + _sel(oh2, kmid) + _sel(oh2, klo)

    e1 = qn - g1
    e2 = qn - g2
    d1 = jnp.sum(e1 * e1, axis=1, keepdims=True)
    d2 = jnp.sum(e2 * e2, axis=1, keepdims=True)
    take2 = (d2 < d1) | ((d2 == d1) & (idx2 < idx1))
    diff = jnp.where(take2, e2, e1)
    d2q = diff * diff
    out_ref[...] = jnp.sum(d2q * d2q, axis=1, keepdims=True)


def kernel(query, keys, train):
    q = query[0]
    b, c, h, w = q.shape
    qr = jnp.transpose(q, (0, 2, 3, 1)).reshape(b * h * w, c)
    loss = pl.pallas_call(
        _hotmap_kernel,
        grid=(_N // _BN,),
        in_specs=[
            pl.BlockSpec((_BN, _D), lambda i: (i, 0)),
            pl.BlockSpec((_M, _D), lambda i: (0, 0)),
        ],
        out_specs=pl.BlockSpec((_BN, 1), lambda i: (i, 0)),
        out_shape=jax.ShapeDtypeStruct((_N, 1), jnp.float32),
        scratch_shapes=[
            pltpu.VMEM((1, _M), jnp.float32),
            pltpu.VMEM((_M, _D), jnp.float32),
            pltpu.VMEM((_M, _D), jnp.float32),
            pltpu.VMEM((_M, _D), jnp.float32),
        ],
    )(qr, keys[0])
    return loss.reshape(b, h, w, 1)
